# baseline (speedup 1.0000x reference)
import functools

import jax
import jax.numpy as jnp
from jax import lax
from jax.experimental import pallas as pl
from jax.experimental.pallas import tpu as pltpu
from jax.experimental.pallas import tpu_sc as plsc

N_NODES = 10000
D = 128
B = 320000
NC, NS, L = 2, 16, 16
NW = NC * NS
EPW = B // NW
C = 80
NCH = EPW // C
G = C // L

_mesh = plsc.VectorSubcoreMesh(core_axis_name="c", subcore_axis_name="s")


@functools.partial(
    pl.kernel,
    out_type=jax.ShapeDtypeStruct((B,), jnp.float32),
    mesh=_mesh,
    scratch_types=[
        pltpu.VMEM((EPW,), jnp.int32),
        pltpu.VMEM((EPW,), jnp.int32),
        pltpu.VMEM((C, D), jnp.float32),
        pltpu.VMEM((C, D), jnp.float32),
        pltpu.VMEM((EPW,), jnp.float32),
        pltpu.SemaphoreType.DMA,
    ],
    compiler_params=pltpu.CompilerParams(needs_layout_passes=False),
)
def _link_classifier(table, src_idx, dst_idx, out_hbm,
                     idx_s, idx_d, rows_s, rows_d, out_v, sem):
    wid = lax.axis_index("s") * NC + lax.axis_index("c")
    base = wid * EPW

    pltpu.sync_copy(src_idx.at[pl.ds(base, EPW)], idx_s)
    pltpu.sync_copy(dst_idx.at[pl.ds(base, EPW)], idx_d)

    lanes = lax.iota(jnp.int32, L)

    def chunk_body(i, carry):
        co = i * C
        pltpu.async_copy(table.at[idx_s.at[pl.ds(co, C)]], rows_s, sem).wait()
        pltpu.async_copy(table.at[idx_d.at[pl.ds(co, C)]], rows_d, sem).wait()

        def j_body(j, accs):
            col = jnp.zeros((L,), jnp.int32) + j
            out = []
            for g in range(G):
                rid = lanes + (g * L)
                a = plsc.load_gather(rows_s, [rid, col])
                b = plsc.load_gather(rows_d, [rid, col])
                out.append(accs[g] + a * b)
            return tuple(out)

        zero = jnp.zeros((L,), jnp.float32)
        accs = lax.fori_loop(0, D, j_body, (zero,) * G)
        for g in range(G):
            out_v[pl.ds(co + g * L, L)] = accs[g]
        return carry

    lax.fori_loop(0, NCH, chunk_body, 0)
    pltpu.sync_copy(out_v, out_hbm.at[pl.ds(base, EPW)])


@jax.jit
def kernel(embedding, edge_label_index):
    idx = edge_label_index.astype(jnp.int32)
    return _link_classifier(embedding, idx[0], idx[1])


# speedup vs baseline: 1.3097x; 1.3097x over previous
import functools

import jax
import jax.numpy as jnp
from jax import lax
from jax.experimental import pallas as pl
from jax.experimental.pallas import tpu as pltpu
from jax.experimental.pallas import tpu_sc as plsc

N_NODES = 10000
D = 128
B = 320000
NC, NS, L = 2, 16, 16
NW = NC * NS
EPW = B // NW
C = 80
NCH = EPW // C
G = C // L
UNROLL = 8

_mesh = plsc.VectorSubcoreMesh(core_axis_name="c", subcore_axis_name="s")


@functools.partial(
    pl.kernel,
    out_type=jax.ShapeDtypeStruct((B,), jnp.float32),
    mesh=_mesh,
    scratch_types=[
        pltpu.VMEM((EPW,), jnp.int32),
        pltpu.VMEM((EPW,), jnp.int32),
        pltpu.VMEM((C, D), jnp.float32),
        pltpu.VMEM((C, D), jnp.float32),
        pltpu.VMEM((C, D), jnp.float32),
        pltpu.VMEM((C, D), jnp.float32),
        pltpu.VMEM((EPW,), jnp.float32),
        pltpu.SemaphoreType.DMA,
        pltpu.SemaphoreType.DMA,
    ],
    compiler_params=pltpu.CompilerParams(
        needs_layout_passes=False,
        disable_bounds_checks=True,
    ),
)
def _link_classifier(table, src_idx, dst_idx, out_hbm,
                     idx_s, idx_d, rs0, rs1, rd0, rd1, out_v, sem0, sem1):
    wid = lax.axis_index("s") * NC + lax.axis_index("c")
    base = wid * EPW

    pltpu.sync_copy(src_idx.at[pl.ds(base, EPW)], idx_s)
    pltpu.sync_copy(dst_idx.at[pl.ds(base, EPW)], idx_d)

    rows = ((rs0, rd0, sem0), (rs1, rd1, sem1))
    lanes = lax.iota(jnp.int32, L)

    def start(i, slot):
        rs, rd, sem = rows[slot]
        co = i * C
        pltpu.make_async_copy(table.at[idx_s.at[pl.ds(co, C)]], rs, sem).start()
        pltpu.make_async_copy(table.at[idx_d.at[pl.ds(co, C)]], rd, sem).start()

    def wait(i, slot):
        rs, rd, sem = rows[slot]
        co = i * C
        pltpu.make_async_copy(table.at[idx_s.at[pl.ds(co, C)]], rs, sem).wait()
        pltpu.make_async_copy(table.at[idx_d.at[pl.ds(co, C)]], rd, sem).wait()

    def compute(i, slot):
        rs, rd, _ = rows[slot]
        co = i * C

        def j_body(j16, accs):
            jbase = j16 * UNROLL
            accs = list(accs)
            for jj in range(UNROLL):
                col = jnp.zeros((L,), jnp.int32) + (jbase + jj)
                for g in range(G):
                    rid = lanes + (g * L)
                    a = plsc.load_gather(rs, [rid, col])
                    b = plsc.load_gather(rd, [rid, col])
                    accs[g] = accs[g] + a * b
            return tuple(accs)

        zero = jnp.zeros((L,), jnp.float32)
        accs = lax.fori_loop(0, D // UNROLL, j_body, (zero,) * G)
        for g in range(G):
            out_v[pl.ds(co + g * L, L)] = accs[g]

    start(0, 0)
    start(1, 1)

    def body2(m, carry):
        i0 = 2 * m
        wait(i0, 0)
        compute(i0, 0)
        start(i0 + 2, 0)
        i1 = i0 + 1
        wait(i1, 1)
        compute(i1, 1)

        @pl.when(m < (NCH // 2) - 1)
        def _():
            start(i1 + 2, 1)
        return carry

    lax.fori_loop(0, NCH // 2, body2, 0)
    wait(NCH - 1, 0)
    compute(NCH - 1, 0)

    pltpu.sync_copy(out_v, out_hbm.at[pl.ds(base, EPW)])


@jax.jit
def kernel(embedding, edge_label_index):
    idx = edge_label_index.astype(jnp.int32)
    return _link_classifier(embedding, idx[0], idx[1])


# speedup vs baseline: 5.1516x; 3.9334x over previous
import functools

import jax
import jax.numpy as jnp
from jax import lax
from jax.experimental import pallas as pl
from jax.experimental.pallas import tpu as pltpu
from jax.experimental.pallas import tpu_sc as plsc

N_NODES = 10000
D = 128
B = 320000
NC, NS, L = 2, 16, 16
NW = NC * NS
EPW = B // NW
C = 80
NCH = EPW // C
G = C // L
UNROLL = 8

_mesh = plsc.VectorSubcoreMesh(core_axis_name="c", subcore_axis_name="s")


@functools.partial(
    pl.kernel,
    out_type=jax.ShapeDtypeStruct((B,), jnp.float32),
    mesh=_mesh,
    scratch_types=[
        pltpu.VMEM((EPW,), jnp.int32),
        pltpu.VMEM((EPW,), jnp.int32),
        pltpu.VMEM((C, D), jnp.float32),
        pltpu.VMEM((C, D), jnp.float32),
        pltpu.VMEM((C, D), jnp.float32),
        pltpu.VMEM((C, D), jnp.float32),
        pltpu.VMEM((EPW,), jnp.float32),
        pltpu.SemaphoreType.DMA,
        pltpu.SemaphoreType.DMA,
    ],
    compiler_params=pltpu.CompilerParams(
        needs_layout_passes=False,
        disable_bounds_checks=True,
    ),
)
def _link_classifier(table, src_idx, dst_idx, out_hbm,
                     idx_s, idx_d, rs0, rs1, rd0, rd1, out_v, sem0, sem1):
    wid = lax.axis_index("s") * NC + lax.axis_index("c")
    base = wid * EPW

    pltpu.sync_copy(src_idx.at[pl.ds(base, EPW)], idx_s)
    pltpu.sync_copy(dst_idx.at[pl.ds(base, EPW)], idx_d)

    rows = ((rs0, rd0, sem0), (rs1, rd1, sem1))
    lanes = lax.iota(jnp.int32, L)

    def start(i, slot):
        rs, rd, sem = rows[slot]
        co = i * C
        pltpu.make_async_copy(table.at[idx_s.at[pl.ds(co, C)]], rs, sem).start()
        pltpu.make_async_copy(table.at[idx_d.at[pl.ds(co, C)]], rd, sem).start()

    def wait(i, slot):
        rs, rd, sem = rows[slot]
        co = i * C
        pltpu.make_async_copy(table.at[idx_s.at[pl.ds(co, C)]], rs, sem).wait()
        pltpu.make_async_copy(table.at[idx_d.at[pl.ds(co, C)]], rd, sem).wait()

    idx15 = jnp.full((L, 1), L - 1, jnp.int32)
    _gd = lax.GatherDimensionNumbers(
        offset_dims=(), collapsed_slice_dims=(0,), start_index_map=(0,))

    def _bcast_last(v):
        return lax.gather(v, idx15, _gd, slice_sizes=(1,),
                          mode=lax.GatherScatterMode.PROMISE_IN_BOUNDS)
    onehots = [
        (lax.iota(jnp.int32, L) == ee).astype(jnp.float32)
        for ee in range(L)
    ]

    def compute(i, slot):
        rs, rd, _ = rows[slot]
        co = i * C

        def e_body(eb, carry):
            contribs = []
            for ee in range(L):
                e = eb * L + ee
                prods = []
                for d in range(D // L):
                    a = rs[e, pl.ds(d * L, L)]
                    b = rd[e, pl.ds(d * L, L)]
                    prods.append(a * b)
                while len(prods) > 1:
                    prods = [x + y for x, y in zip(prods[::2], prods[1::2])]
                cum = jnp.cumsum(prods[0])
                bcast = _bcast_last(cum)
                contribs.append(bcast * onehots[ee])
            while len(contribs) > 1:
                contribs = [x + y for x, y in zip(contribs[::2], contribs[1::2])]
            out_v[pl.ds(co + eb * L, L)] = contribs[0]
            return carry

        lax.fori_loop(0, C // L, e_body, 0)

    start(0, 0)
    start(1, 1)

    def body2(m, carry):
        i0 = 2 * m
        wait(i0, 0)
        compute(i0, 0)
        start(i0 + 2, 0)
        i1 = i0 + 1
        wait(i1, 1)
        compute(i1, 1)

        @pl.when(m < (NCH // 2) - 1)
        def _():
            start(i1 + 2, 1)
        return carry

    lax.fori_loop(0, NCH // 2, body2, 0)
    wait(NCH - 1, 0)
    compute(NCH - 1, 0)

    pltpu.sync_copy(out_v, out_hbm.at[pl.ds(base, EPW)])


@jax.jit
def kernel(embedding, edge_label_index):
    idx = edge_label_index.astype(jnp.int32)
    return _link_classifier(embedding, idx[0], idx[1])
